# baseline (speedup 1.0000x reference)
import functools

import jax
import jax.numpy as jnp
from jax import lax
from jax.experimental import pallas as pl
from jax.experimental.pallas import tpu as pltpu
from jax.experimental.pallas import tpu_sc as plsc

N = 10000
E = 320000
DI = 128
H = 64
NC = 2
NS = 16
NW = NC * NS
K = 128
NCH = E // K
CH_T = 80
NCHP = CH_T * NW
PAIRS = CH_T // 2
SPAN = 640
NP = NS * SPAN

_mesh = plsc.VectorSubcoreMesh(core_axis_name="c", subcore_axis_name="s")


def _striped(src_fn, dst_fn, s):
    off = pl.multiple_of(s * SPAN, 128)
    pltpu.sync_copy(src_fn(off, SPAN), dst_fn(off, SPAN))


@functools.partial(
    pl.kernel,
    out_type=jax.ShapeDtypeStruct((NC, NP), jnp.float32),
    mesh=_mesh,
    scratch_types=[
        pltpu.VMEM((CH_T, K), jnp.int32),
        pltpu.VMEM((K,), jnp.float32),
        pltpu.SemaphoreType.DMA,
        pltpu.VMEM_SHARED((NP,), jnp.float32),
    ],
    compiler_params=pltpu.CompilerParams(use_tc_tiling_on_sc=False),
)
def _deg_kernel(dst2_h, z1_h, deg_o, dst_i, ones_v, dsem, deg_sh):
    c = lax.axis_index("c")
    s = lax.axis_index("s")
    wid = c * NS + s
    for i in range(K // 16):
        ones_v[pl.ds(i * 16, 16)] = jnp.ones((16,), jnp.float32)
    ibase = pl.multiple_of(wid * CH_T, 8)
    pltpu.sync_copy(dst2_h.at[pl.ds(ibase, CH_T)], dst_i)
    _striped(lambda o, n: z1_h.at[pl.ds(o, n)],
             lambda o, n: deg_sh.at[pl.ds(o, n)], s)
    plsc.subcore_barrier()

    def fire(j, carry):
        pltpu.async_copy(ones_v, deg_sh.at[dst_i.at[j]], dsem, add=True)
        return carry

    lax.fori_loop(0, CH_T, fire, 0)

    def drain(j, carry):
        pltpu.make_async_copy(ones_v, deg_sh.at[dst_i.at[j]], dsem).wait()
        return carry

    lax.fori_loop(0, CH_T, drain, 0)
    plsc.subcore_barrier()
    _striped(lambda o, n: deg_sh.at[pl.ds(o, n)],
             lambda o, n: deg_o.at[c, pl.ds(o, n)], s)


NBUF = 4


def _make_edge_pass(with_s):
    out_type = [jax.ShapeDtypeStruct((NC, NP, H), jnp.float32)]
    scratch = (
        [pltpu.VMEM((CH_T, K), jnp.int32)] * 2
        + [pltpu.VMEM((K, H), jnp.float32)] * NBUF
        + [pltpu.SemaphoreType.DMA] * (2 * NBUF)
        + [pltpu.VMEM_SHARED((NP, H), jnp.float32)]
    )
    if with_s:
        out_type.append(jax.ShapeDtypeStruct((NC, NP), jnp.float32))
        scratch += (
            [pltpu.VMEM((K,), jnp.float32)] * NBUF
            + [pltpu.SemaphoreType.DMA] * (2 * NBUF)
            + [pltpu.VMEM_SHARED((NP,), jnp.float32)]
        )

    def body(*args):
        if with_s:
            (hh_h, src2_h, dst2_h, dinv_h, z2_h, z1_h, agg_o, s_o), rest = args[:8], args[8:]
        else:
            (hh_h, src2_h, dst2_h, z2_h, agg_o), rest = args[:5], args[5:]
        src_i, dst_i = rest[0], rest[1]
        rows = rest[2:2 + NBUF]
        gsem = rest[2 + NBUF:2 + 2 * NBUF]
        csem = rest[2 + 2 * NBUF:2 + 3 * NBUF]
        agg_sh = rest[2 + 3 * NBUF]
        if with_s:
            r2 = rest[3 + 3 * NBUF:]
            vals = r2[:NBUF]
            vgsem = r2[NBUF:2 * NBUF]
            vcsem = r2[2 * NBUF:3 * NBUF]
            s_sh = r2[3 * NBUF]
        c = lax.axis_index("c")
        s = lax.axis_index("s")
        wid = c * NS + s
        ibase = pl.multiple_of(wid * CH_T, 8)
        pltpu.sync_copy(src2_h.at[pl.ds(ibase, CH_T)], src_i)
        pltpu.sync_copy(dst2_h.at[pl.ds(ibase, CH_T)], dst_i)

        def g_fire(j, b):
            pltpu.async_copy(hh_h.at[src_i.at[j]], rows[b], gsem[b])
            if with_s:
                pltpu.async_copy(dinv_h.at[dst_i.at[j]], vals[b], vgsem[b])

        def g_wait(j, b):
            pltpu.make_async_copy(hh_h.at[src_i.at[j]], rows[b], gsem[b]).wait()
            if with_s:
                pltpu.make_async_copy(dinv_h.at[dst_i.at[j]], vals[b],
                                      vgsem[b]).wait()

        def c_fire(j, b):
            pltpu.async_copy(rows[b], agg_sh.at[dst_i.at[j]], csem[b], add=True)
            if with_s:
                pltpu.async_copy(vals[b], s_sh.at[src_i.at[j]], vcsem[b],
                                 add=True)

        def c_wait(j, b):
            pltpu.make_async_copy(rows[b], agg_sh.at[dst_i.at[j]],
                                  csem[b]).wait()
            if with_s:
                pltpu.make_async_copy(vals[b], s_sh.at[src_i.at[j]],
                                      vcsem[b]).wait()

        g_fire(0, 0)
        g_fire(1, 1)
        _striped(lambda o, n: z2_h.at[pl.ds(o, n)],
                 lambda o, n: agg_sh.at[pl.ds(o, n)], s)
        if with_s:
            _striped(lambda o, n: z1_h.at[pl.ds(o, n)],
                     lambda o, n: s_sh.at[pl.ds(o, n)], s)
        plsc.subcore_barrier()

        def group(g, carry):
            for b in range(NBUF):
                j = NBUF * g + b
                g_wait(j, b)
                c_fire(j, b)
                b2 = (b + 2) % NBUF

                @pl.when(j >= 2)
                def _():
                    c_wait(j - 2, b2)

                @pl.when(j + 2 < CH_T)
                def _():
                    g_fire(j + 2, b2)
            return carry

        lax.fori_loop(0, CH_T // NBUF, group, 0)
        c_wait(CH_T - 2, (CH_T - 2) % NBUF)
        c_wait(CH_T - 1, (CH_T - 1) % NBUF)
        plsc.subcore_barrier()
        _striped(lambda o, n: agg_sh.at[pl.ds(o, n)],
                 lambda o, n: agg_o.at[c, pl.ds(o, n)], s)
        if with_s:
            _striped(lambda o, n: s_sh.at[pl.ds(o, n)],
                     lambda o, n: s_o.at[c, pl.ds(o, n)], s)

    return pl.kernel(
        body,
        out_type=tuple(out_type) if with_s else out_type[0],
        mesh=_mesh,
        scratch_types=scratch,
        compiler_params=pltpu.CompilerParams(use_tc_tiling_on_sc=False),
    )


_edge_pass_s = _make_edge_pass(True)
_edge_pass = _make_edge_pass(False)


_BR = 80
_GRID = N // _BR


def _tc1_body(deg_ref, x_ref, w_ref, hh_ref, dinv_ref):
    d = deg_ref[...]
    dinv = lax.rsqrt(d[:, 0:1] + d[:, 1:2] + 1.0)
    hh_ref[...] = dinv * jnp.dot(x_ref[...], w_ref[...],
                                 preferred_element_type=jnp.float32)
    dinv_ref[...] = dinv


def _tc1(degT, x, W1):
    return pl.pallas_call(
        _tc1_body,
        grid=(_GRID,),
        in_specs=[
            pl.BlockSpec((_BR, NC), lambda i: (i, 0)),
            pl.BlockSpec((_BR, DI), lambda i: (i, 0)),
            pl.BlockSpec((DI, H), lambda i: (0, 0)),
        ],
        out_specs=[
            pl.BlockSpec((_BR, H), lambda i: (i, 0)),
            pl.BlockSpec((_BR, 1), lambda i: (i, 0)),
        ],
        out_shape=[
            jax.ShapeDtypeStruct((NP, H), jnp.float32),
            jax.ShapeDtypeStruct((NP, 1), jnp.float32),
        ],
    )(degT, x, W1)


def _tc2_body(agg_ref, hh_ref, dinv_ref, b_ref, w_ref, out_ref):
    a = agg_ref[0] + agg_ref[1]
    dinv = dinv_ref[...]
    h = jnp.maximum(dinv * (a + hh_ref[...]) + b_ref[...], 0.0)
    out_ref[...] = dinv * jnp.dot(h, w_ref[...],
                                  preferred_element_type=jnp.float32)


def _tc2(agg, hh, dinv, b, W):
    return pl.pallas_call(
        _tc2_body,
        grid=(_GRID,),
        in_specs=[
            pl.BlockSpec((NC, _BR, H), lambda i: (0, i, 0)),
            pl.BlockSpec((_BR, H), lambda i: (i, 0)),
            pl.BlockSpec((_BR, 1), lambda i: (i, 0)),
            pl.BlockSpec((1, H), lambda i: (0, 0)),
            pl.BlockSpec((H, H), lambda i: (0, 0)),
        ],
        out_specs=pl.BlockSpec((_BR, H), lambda i: (i, 0)),
        out_shape=jax.ShapeDtypeStruct((NP, H), jnp.float32),
    )(agg, hh, dinv, b, W)


def _tc3_body(agg_ref, hh_ref, dinv_ref, s_ref, b_ref,
              w3_ref, b3_ref, wc1_ref, bc1_ref, wc2_ref, bc2_ref,
              u_ref, o_ref):
    a = agg_ref[0] + agg_ref[1]
    dinv = dinv_ref[...]
    h2 = jnp.maximum(dinv * (a + hh_ref[...]) + b_ref[...], 0.0)
    sv = s_ref[...]
    w = dinv * (sv[:, 0:1] + sv[:, 1:2] + dinv)

    @pl.when(pl.program_id(0) == 0)
    def _():
        u_ref[...] = jnp.zeros_like(u_ref)

    u_ref[...] += jnp.sum(w * h2, axis=0, keepdims=True)

    @pl.when(pl.program_id(0) == _GRID - 1)
    def _():
        g = jnp.dot(u_ref[...], w3_ref[...],
                    preferred_element_type=jnp.float32) * (1.0 / N) + b3_ref[...]
        z = jnp.maximum(jnp.dot(g, wc1_ref[...],
                                preferred_element_type=jnp.float32)
                        + bc1_ref[...], 0.0)
        o_ref[...] = jnp.dot(z, wc2_ref[...],
                             preferred_element_type=jnp.float32) + bc2_ref[...]


def _tc3(agg, hh, dinv, sT, b, W3, b3, Wc1, bc1, Wc2, bc2):
    full = lambda i: (0, 0)
    return pl.pallas_call(
        _tc3_body,
        grid=(_GRID,),
        in_specs=[
            pl.BlockSpec((NC, _BR, H), lambda i: (0, i, 0)),
            pl.BlockSpec((_BR, H), lambda i: (i, 0)),
            pl.BlockSpec((_BR, 1), lambda i: (i, 0)),
            pl.BlockSpec((_BR, NC), lambda i: (i, 0)),
            pl.BlockSpec((1, H), full),
            pl.BlockSpec((H, H), full),
            pl.BlockSpec((1, H), full),
            pl.BlockSpec((H, H // 2), full),
            pl.BlockSpec((1, H // 2), full),
            pl.BlockSpec((H // 2, 5), full),
            pl.BlockSpec((1, 5), full),
        ],
        out_specs=[
            pl.BlockSpec((1, H), full),
            pl.BlockSpec((1, 5), full),
        ],
        out_shape=[
            jax.ShapeDtypeStruct((1, H), jnp.float32),
            jax.ShapeDtypeStruct((1, 5), jnp.float32),
        ],
    )(agg, hh, dinv, sT, b, W3, b3, Wc1, bc1, Wc2, bc2)


@jax.jit
def kernel(x, edge_index, W1, b1, W2, b2, W3, b3, Wc1, bc1, Wc2, bc2):
    ei = edge_index.astype(jnp.int32)
    pad_idx = N + jnp.arange(NCHP * K - E, dtype=jnp.int32) % (NP - N)
    pad_idx = jnp.broadcast_to(pad_idx, (2, NCHP * K - E))
    ei2 = jnp.concatenate([ei, pad_idx], axis=1).reshape(2, NCHP, K)
    src2, dst2 = ei2[0], ei2[1]
    z1 = jnp.zeros((NP,), jnp.float32)
    z2 = jnp.zeros((NP, H), jnp.float32)

    degp = _deg_kernel(dst2, z1)
    hh1p, dinv = _tc1(degp.T, x, W1)
    agg1, sp = _edge_pass_s(hh1p, src2, dst2, dinv[:, 0], z2, z1)
    hh2p = _tc2(agg1, hh1p, dinv, b1.reshape(1, H), W2)
    agg2 = _edge_pass(hh2p, src2, dst2, z2)
    _, out = _tc3(agg2, hh2p, dinv, sp.T, b2.reshape(1, H),
                  W3, b3.reshape(1, H), Wc1, bc1.reshape(1, H // 2),
                  Wc2, bc2.reshape(1, 5))
    return out


# speedup vs baseline: 1.6741x; 1.6741x over previous
import functools

import jax
import jax.numpy as jnp
from jax import lax
from jax.experimental import pallas as pl
from jax.experimental.pallas import tpu as pltpu
from jax.experimental.pallas import tpu_sc as plsc

N = 10000
E = 320000
DI = 128
H = 64
NC = 2
NS = 16
NW = NC * NS
K = 128
NCH = E // K
CH_T = 80
NCHP = CH_T * NW
PAIRS = CH_T // 2
SPAN = 640
NP = NS * SPAN

_mesh = plsc.VectorSubcoreMesh(core_axis_name="c", subcore_axis_name="s")


def _striped(src_fn, dst_fn, s):
    off = pl.multiple_of(s * SPAN, 128)
    pltpu.sync_copy(src_fn(off, SPAN), dst_fn(off, SPAN))


@functools.partial(
    pl.kernel,
    out_type=jax.ShapeDtypeStruct((NC, NP), jnp.float32),
    mesh=_mesh,
    scratch_types=[
        pltpu.VMEM((CH_T, K), jnp.int32),
        pltpu.VMEM((K,), jnp.float32),
        pltpu.SemaphoreType.DMA,
        pltpu.VMEM_SHARED((NP,), jnp.float32),
    ],
    compiler_params=pltpu.CompilerParams(use_tc_tiling_on_sc=False),
)
def _deg_kernel(dst2_h, z1_h, deg_o, dst_i, ones_v, dsem, deg_sh):
    c = lax.axis_index("c")
    s = lax.axis_index("s")
    wid = c * NS + s
    for i in range(K // 16):
        ones_v[pl.ds(i * 16, 16)] = jnp.ones((16,), jnp.float32)
    ibase = pl.multiple_of(wid * CH_T, 8)
    pltpu.sync_copy(dst2_h.at[pl.ds(ibase, CH_T)], dst_i)
    _striped(lambda o, n: z1_h.at[pl.ds(o, n)],
             lambda o, n: deg_sh.at[pl.ds(o, n)], s)
    plsc.subcore_barrier()

    def fire(j, carry):
        pltpu.async_copy(ones_v, deg_sh.at[dst_i.at[j]], dsem, add=True)
        return carry

    lax.fori_loop(0, CH_T, fire, 0)

    def drain(j, carry):
        pltpu.make_async_copy(ones_v, deg_sh.at[dst_i.at[j]], dsem).wait()
        return carry

    lax.fori_loop(0, CH_T, drain, 0)
    plsc.subcore_barrier()
    _striped(lambda o, n: deg_sh.at[pl.ds(o, n)],
             lambda o, n: deg_o.at[c, pl.ds(o, n)], s)


NBUF = 4


def _make_edge_pass(with_s):
    out_type = [jax.ShapeDtypeStruct((NC, NP, H), jnp.float32)]
    scratch = (
        [pltpu.VMEM((CH_T, K), jnp.int32)] * 2
        + [pltpu.VMEM((K, H), jnp.float32)] * NBUF
        + [pltpu.SemaphoreType.DMA] * (2 * NBUF)
        + [pltpu.VMEM_SHARED((NP, H), jnp.float32)]
    )
    if with_s:
        out_type.append(jax.ShapeDtypeStruct((NC, NP), jnp.float32))
        scratch += (
            [pltpu.VMEM((K,), jnp.float32)] * NBUF
            + [pltpu.SemaphoreType.DMA] * (2 * NBUF)
            + [pltpu.VMEM_SHARED((NP,), jnp.float32)]
        )

    def body(*args):
        if with_s:
            (hh_h, src2_h, dst2_h, dinv_h, z2_h, z1_h, agg_o, s_o), rest = args[:8], args[8:]
        else:
            (hh_h, src2_h, dst2_h, z2_h, agg_o), rest = args[:5], args[5:]
        src_i, dst_i = rest[0], rest[1]
        rows = rest[2:2 + NBUF]
        gsem = rest[2 + NBUF:2 + 2 * NBUF]
        csem = rest[2 + 2 * NBUF:2 + 3 * NBUF]
        agg_sh = rest[2 + 3 * NBUF]
        if with_s:
            r2 = rest[3 + 3 * NBUF:]
            vals = r2[:NBUF]
            vgsem = r2[NBUF:2 * NBUF]
            vcsem = r2[2 * NBUF:3 * NBUF]
            s_sh = r2[3 * NBUF]
        c = lax.axis_index("c")
        s = lax.axis_index("s")
        wid = c * NS + s
        ibase = pl.multiple_of(wid * CH_T, 8)
        pltpu.sync_copy(src2_h.at[pl.ds(ibase, CH_T)], src_i)
        pltpu.sync_copy(dst2_h.at[pl.ds(ibase, CH_T)], dst_i)

        def g_fire(j, b):
            pltpu.async_copy(hh_h.at[src_i.at[j]], rows[b], gsem[b])
            if with_s:
                pltpu.async_copy(dinv_h.at[dst_i.at[j]], vals[b], vgsem[b])

        def g_wait(j, b):
            pltpu.make_async_copy(hh_h.at[src_i.at[j]], rows[b], gsem[b]).wait()
            if with_s:
                pltpu.make_async_copy(dinv_h.at[dst_i.at[j]], vals[b],
                                      vgsem[b]).wait()

        def c_fire(j, b):
            pltpu.async_copy(rows[b], agg_sh.at[dst_i.at[j]], csem[b], add=True)
            if with_s:
                pltpu.async_copy(vals[b], s_sh.at[src_i.at[j]], vcsem[b],
                                 add=True)

        def c_wait(j, b):
            pltpu.make_async_copy(rows[b], agg_sh.at[dst_i.at[j]],
                                  csem[b]).wait()
            if with_s:
                pltpu.make_async_copy(vals[b], s_sh.at[src_i.at[j]],
                                      vcsem[b]).wait()

        g_fire(0, 0)
        g_fire(1, 1)
        _striped(lambda o, n: z2_h.at[pl.ds(o, n)],
                 lambda o, n: agg_sh.at[pl.ds(o, n)], s)
        if with_s:
            _striped(lambda o, n: z1_h.at[pl.ds(o, n)],
                     lambda o, n: s_sh.at[pl.ds(o, n)], s)
        plsc.subcore_barrier()

        def group(g, carry):
            for b in range(NBUF):
                j = NBUF * g + b
                g_wait(j, b)
                c_fire(j, b)
                b2 = (b + 2) % NBUF

                @pl.when(j >= 2)
                def _():
                    c_wait(j - 2, b2)

                @pl.when(j + 2 < CH_T)
                def _():
                    g_fire(j + 2, b2)
            return carry

        lax.fori_loop(0, CH_T // NBUF, group, 0)
        c_wait(CH_T - 2, (CH_T - 2) % NBUF)
        c_wait(CH_T - 1, (CH_T - 1) % NBUF)
        plsc.subcore_barrier()
        _striped(lambda o, n: agg_sh.at[pl.ds(o, n)],
                 lambda o, n: agg_o.at[c, pl.ds(o, n)], s)
        if with_s:
            _striped(lambda o, n: s_sh.at[pl.ds(o, n)],
                     lambda o, n: s_o.at[c, pl.ds(o, n)], s)

    return pl.kernel(
        body,
        out_type=tuple(out_type) if with_s else out_type[0],
        mesh=_mesh,
        scratch_types=scratch,
        compiler_params=pltpu.CompilerParams(use_tc_tiling_on_sc=False),
    )


_edge_pass_s = _make_edge_pass(True)
_edge_pass = _make_edge_pass(False)


_BR = 640
_GRID = NP // _BR


def _tc1_body(deg_ref, x_ref, w_ref, hh_ref, dinv_ref):
    d = deg_ref[...]
    dinv = lax.rsqrt(d[:, 0:1] + d[:, 1:2] + 1.0)
    hh_ref[...] = dinv * jnp.dot(x_ref[...], w_ref[...],
                                 preferred_element_type=jnp.float32)
    dinv_ref[...] = dinv


def _tc1(degT, x, W1):
    return pl.pallas_call(
        _tc1_body,
        grid=(_GRID,),
        in_specs=[
            pl.BlockSpec((_BR, NC), lambda i: (i, 0)),
            pl.BlockSpec((_BR, DI), lambda i: (i, 0)),
            pl.BlockSpec((DI, H), lambda i: (0, 0)),
        ],
        out_specs=[
            pl.BlockSpec((_BR, H), lambda i: (i, 0)),
            pl.BlockSpec((_BR, 1), lambda i: (i, 0)),
        ],
        out_shape=[
            jax.ShapeDtypeStruct((NP, H), jnp.float32),
            jax.ShapeDtypeStruct((NP, 1), jnp.float32),
        ],
    )(degT, x, W1)


def _tc2_body(agg_ref, hh_ref, dinv_ref, b_ref, w_ref, out_ref):
    a = agg_ref[0] + agg_ref[1]
    dinv = dinv_ref[...]
    h = jnp.maximum(dinv * (a + hh_ref[...]) + b_ref[...], 0.0)
    out_ref[...] = dinv * jnp.dot(h, w_ref[...],
                                  preferred_element_type=jnp.float32)


def _tc2(agg, hh, dinv, b, W):
    return pl.pallas_call(
        _tc2_body,
        grid=(_GRID,),
        in_specs=[
            pl.BlockSpec((NC, _BR, H), lambda i: (0, i, 0)),
            pl.BlockSpec((_BR, H), lambda i: (i, 0)),
            pl.BlockSpec((_BR, 1), lambda i: (i, 0)),
            pl.BlockSpec((1, H), lambda i: (0, 0)),
            pl.BlockSpec((H, H), lambda i: (0, 0)),
        ],
        out_specs=pl.BlockSpec((_BR, H), lambda i: (i, 0)),
        out_shape=jax.ShapeDtypeStruct((NP, H), jnp.float32),
    )(agg, hh, dinv, b, W)


def _tc3_body(agg_ref, hh_ref, dinv_ref, s_ref, b_ref,
              w3_ref, b3_ref, wc1_ref, bc1_ref, wc2_ref, bc2_ref,
              u_ref, o_ref):
    a = agg_ref[0] + agg_ref[1]
    dinv = dinv_ref[...]
    h2 = jnp.maximum(dinv * (a + hh_ref[...]) + b_ref[...], 0.0)
    sv = s_ref[...]
    w = dinv * (sv[:, 0:1] + sv[:, 1:2] + dinv)
    rowid = (lax.broadcasted_iota(jnp.int32, (_BR, 1), 0)
             + pl.program_id(0) * _BR)
    w = jnp.where(rowid < N, w, 0.0)

    @pl.when(pl.program_id(0) == 0)
    def _():
        u_ref[...] = jnp.zeros_like(u_ref)

    u_ref[...] += jnp.sum(w * h2, axis=0, keepdims=True)

    @pl.when(pl.program_id(0) == _GRID - 1)
    def _():
        g = jnp.dot(u_ref[...], w3_ref[...],
                    preferred_element_type=jnp.float32) * (1.0 / N) + b3_ref[...]
        z = jnp.maximum(jnp.dot(g, wc1_ref[...],
                                preferred_element_type=jnp.float32)
                        + bc1_ref[...], 0.0)
        o_ref[...] = jnp.dot(z, wc2_ref[...],
                             preferred_element_type=jnp.float32) + bc2_ref[...]


def _tc3(agg, hh, dinv, sT, b, W3, b3, Wc1, bc1, Wc2, bc2):
    full = lambda i: (0, 0)
    return pl.pallas_call(
        _tc3_body,
        grid=(_GRID,),
        in_specs=[
            pl.BlockSpec((NC, _BR, H), lambda i: (0, i, 0)),
            pl.BlockSpec((_BR, H), lambda i: (i, 0)),
            pl.BlockSpec((_BR, 1), lambda i: (i, 0)),
            pl.BlockSpec((_BR, NC), lambda i: (i, 0)),
            pl.BlockSpec((1, H), full),
            pl.BlockSpec((H, H), full),
            pl.BlockSpec((1, H), full),
            pl.BlockSpec((H, H // 2), full),
            pl.BlockSpec((1, H // 2), full),
            pl.BlockSpec((H // 2, 5), full),
            pl.BlockSpec((1, 5), full),
        ],
        out_specs=[
            pl.BlockSpec((1, H), full),
            pl.BlockSpec((1, 5), full),
        ],
        out_shape=[
            jax.ShapeDtypeStruct((1, H), jnp.float32),
            jax.ShapeDtypeStruct((1, 5), jnp.float32),
        ],
    )(agg, hh, dinv, sT, b, W3, b3, Wc1, bc1, Wc2, bc2)


@jax.jit
def kernel(x, edge_index, W1, b1, W2, b2, W3, b3, Wc1, bc1, Wc2, bc2):
    ei = edge_index.astype(jnp.int32)
    pad_idx = N + jnp.arange(NCHP * K - E, dtype=jnp.int32) % (NP - N)
    pad_idx = jnp.broadcast_to(pad_idx, (2, NCHP * K - E))
    ei2 = jnp.concatenate([ei, pad_idx], axis=1).reshape(2, NCHP, K)
    src2, dst2 = ei2[0], ei2[1]
    x_pad = jnp.pad(x, ((0, NP - N), (0, 0)))
    z1 = jnp.zeros((NP,), jnp.float32)
    z2 = jnp.zeros((NP, H), jnp.float32)

    degp = _deg_kernel(dst2, z1)
    hh1p, dinv = _tc1(degp.T, x_pad, W1)
    agg1, sp = _edge_pass_s(hh1p, src2, dst2, dinv[:, 0], z2, z1)
    hh2p = _tc2(agg1, hh1p, dinv, b1.reshape(1, H), W2)
    agg2 = _edge_pass(hh2p, src2, dst2, z2)
    _, out = _tc3(agg2, hh2p, dinv, sp.T, b2.reshape(1, H),
                  W3, b3.reshape(1, H), Wc1, bc1.reshape(1, H // 2),
                  Wc2, bc2.reshape(1, 5))
    return out


# speedup vs baseline: 1.6747x; 1.0003x over previous
import functools

import jax
import jax.numpy as jnp
from jax import lax
from jax.experimental import pallas as pl
from jax.experimental.pallas import tpu as pltpu
from jax.experimental.pallas import tpu_sc as plsc

N = 10000
E = 320000
DI = 128
H = 64
NC = 2
NS = 16
NW = NC * NS
K = 128
NCH = E // K
CH_T = 80
NCHP = CH_T * NW
PAIRS = CH_T // 2
SPAN = 640
NP = NS * SPAN

_mesh = plsc.VectorSubcoreMesh(core_axis_name="c", subcore_axis_name="s")


def _striped(src_fn, dst_fn, s):
    off = pl.multiple_of(s * SPAN, 128)
    pltpu.sync_copy(src_fn(off, SPAN), dst_fn(off, SPAN))


@functools.partial(
    pl.kernel,
    out_type=jax.ShapeDtypeStruct((NC, NP), jnp.float32),
    mesh=_mesh,
    scratch_types=[
        pltpu.VMEM((CH_T, K), jnp.int32),
        pltpu.VMEM((K,), jnp.float32),
        pltpu.SemaphoreType.DMA,
        pltpu.VMEM_SHARED((NP,), jnp.float32),
    ],
    compiler_params=pltpu.CompilerParams(use_tc_tiling_on_sc=False),
)
def _deg_kernel(dst2_h, z1_h, deg_o, dst_i, ones_v, dsem, deg_sh):
    c = lax.axis_index("c")
    s = lax.axis_index("s")
    wid = c * NS + s
    for i in range(K // 16):
        ones_v[pl.ds(i * 16, 16)] = jnp.ones((16,), jnp.float32)
    ibase = pl.multiple_of(wid * CH_T, 8)
    pltpu.sync_copy(dst2_h.at[pl.ds(ibase, CH_T)], dst_i)
    _striped(lambda o, n: z1_h.at[pl.ds(o, n)],
             lambda o, n: deg_sh.at[pl.ds(o, n)], s)
    plsc.subcore_barrier()

    def fire(j, carry):
        pltpu.async_copy(ones_v, deg_sh.at[dst_i.at[j]], dsem, add=True)
        return carry

    lax.fori_loop(0, CH_T, fire, 0)

    def drain(j, carry):
        pltpu.make_async_copy(ones_v, deg_sh.at[dst_i.at[j]], dsem).wait()
        return carry

    lax.fori_loop(0, CH_T, drain, 0)
    plsc.subcore_barrier()
    _striped(lambda o, n: deg_sh.at[pl.ds(o, n)],
             lambda o, n: deg_o.at[c, pl.ds(o, n)], s)


NBUF = 4
LEAD = 2


def _make_edge_pass(with_s):
    out_type = [jax.ShapeDtypeStruct((NC, NP, H), jnp.float32)]
    scratch = (
        [pltpu.VMEM((CH_T, K), jnp.int32)] * 2
        + [pltpu.VMEM((K, H), jnp.float32)] * NBUF
        + [pltpu.SemaphoreType.DMA] * (2 * NBUF)
        + [pltpu.VMEM_SHARED((NP, H), jnp.float32)]
    )
    if with_s:
        out_type.append(jax.ShapeDtypeStruct((NC, NP), jnp.float32))
        scratch += (
            [pltpu.VMEM((K,), jnp.float32)] * NBUF
            + [pltpu.SemaphoreType.DMA] * (2 * NBUF)
            + [pltpu.VMEM_SHARED((NP,), jnp.float32)]
        )

    def body(*args):
        if with_s:
            (hh_h, src2_h, dst2_h, dinv_h, z2_h, z1_h, agg_o, s_o), rest = args[:8], args[8:]
        else:
            (hh_h, src2_h, dst2_h, z2_h, agg_o), rest = args[:5], args[5:]
        src_i, dst_i = rest[0], rest[1]
        rows = rest[2:2 + NBUF]
        gsem = rest[2 + NBUF:2 + 2 * NBUF]
        csem = rest[2 + 2 * NBUF:2 + 3 * NBUF]
        agg_sh = rest[2 + 3 * NBUF]
        if with_s:
            r2 = rest[3 + 3 * NBUF:]
            vals = r2[:NBUF]
            vgsem = r2[NBUF:2 * NBUF]
            vcsem = r2[2 * NBUF:3 * NBUF]
            s_sh = r2[3 * NBUF]
        c = lax.axis_index("c")
        s = lax.axis_index("s")
        wid = c * NS + s
        ibase = pl.multiple_of(wid * CH_T, 8)
        pltpu.sync_copy(src2_h.at[pl.ds(ibase, CH_T)], src_i)
        pltpu.sync_copy(dst2_h.at[pl.ds(ibase, CH_T)], dst_i)

        def g_fire(j, b):
            pltpu.async_copy(hh_h.at[src_i.at[j]], rows[b], gsem[b])
            if with_s:
                pltpu.async_copy(dinv_h.at[dst_i.at[j]], vals[b], vgsem[b])

        def g_wait(j, b):
            pltpu.make_async_copy(hh_h.at[src_i.at[j]], rows[b], gsem[b]).wait()
            if with_s:
                pltpu.make_async_copy(dinv_h.at[dst_i.at[j]], vals[b],
                                      vgsem[b]).wait()

        def c_fire(j, b):
            pltpu.async_copy(rows[b], agg_sh.at[dst_i.at[j]], csem[b], add=True)
            if with_s:
                pltpu.async_copy(vals[b], s_sh.at[src_i.at[j]], vcsem[b],
                                 add=True)

        def c_wait(j, b):
            pltpu.make_async_copy(rows[b], agg_sh.at[dst_i.at[j]],
                                  csem[b]).wait()
            if with_s:
                pltpu.make_async_copy(vals[b], s_sh.at[src_i.at[j]],
                                      vcsem[b]).wait()

        for j0 in range(LEAD):
            g_fire(j0, j0)
        _striped(lambda o, n: z2_h.at[pl.ds(o, n)],
                 lambda o, n: agg_sh.at[pl.ds(o, n)], s)
        if with_s:
            _striped(lambda o, n: z1_h.at[pl.ds(o, n)],
                     lambda o, n: s_sh.at[pl.ds(o, n)], s)
        plsc.subcore_barrier()

        def group(g, carry):
            for b in range(NBUF):
                j = NBUF * g + b
                g_wait(j, b)
                c_fire(j, b)
                b2 = (b + LEAD) % NBUF

                @pl.when(j >= NBUF - LEAD)
                def _():
                    c_wait(j + LEAD - NBUF, b2)

                @pl.when(j + LEAD < CH_T)
                def _():
                    g_fire(j + LEAD, b2)
            return carry

        lax.fori_loop(0, CH_T // NBUF, group, 0)
        for t in range(NBUF - LEAD):
            jt = CH_T - (NBUF - LEAD) + t
            c_wait(jt, jt % NBUF)
        plsc.subcore_barrier()
        _striped(lambda o, n: agg_sh.at[pl.ds(o, n)],
                 lambda o, n: agg_o.at[c, pl.ds(o, n)], s)
        if with_s:
            _striped(lambda o, n: s_sh.at[pl.ds(o, n)],
                     lambda o, n: s_o.at[c, pl.ds(o, n)], s)

    return pl.kernel(
        body,
        out_type=tuple(out_type) if with_s else out_type[0],
        mesh=_mesh,
        scratch_types=scratch,
        compiler_params=pltpu.CompilerParams(use_tc_tiling_on_sc=False),
    )


_edge_pass_s = _make_edge_pass(True)
_edge_pass = _make_edge_pass(False)


_BR = 640
_GRID = NP // _BR


def _tc1_body(deg_ref, x_ref, w_ref, hh_ref, dinv_ref):
    d = deg_ref[...]
    dinv = lax.rsqrt(d[:, 0:1] + d[:, 1:2] + 1.0)
    hh_ref[...] = dinv * jnp.dot(x_ref[...], w_ref[...],
                                 preferred_element_type=jnp.float32)
    dinv_ref[...] = dinv


def _tc1(degT, x, W1):
    return pl.pallas_call(
        _tc1_body,
        grid=(_GRID,),
        in_specs=[
            pl.BlockSpec((_BR, NC), lambda i: (i, 0)),
            pl.BlockSpec((_BR, DI), lambda i: (i, 0)),
            pl.BlockSpec((DI, H), lambda i: (0, 0)),
        ],
        out_specs=[
            pl.BlockSpec((_BR, H), lambda i: (i, 0)),
            pl.BlockSpec((_BR, 1), lambda i: (i, 0)),
        ],
        out_shape=[
            jax.ShapeDtypeStruct((NP, H), jnp.float32),
            jax.ShapeDtypeStruct((NP, 1), jnp.float32),
        ],
    )(degT, x, W1)


def _tc2_body(agg_ref, hh_ref, dinv_ref, b_ref, w_ref, out_ref):
    a = agg_ref[0] + agg_ref[1]
    dinv = dinv_ref[...]
    h = jnp.maximum(dinv * (a + hh_ref[...]) + b_ref[...], 0.0)
    out_ref[...] = dinv * jnp.dot(h, w_ref[...],
                                  preferred_element_type=jnp.float32)


def _tc2(agg, hh, dinv, b, W):
    return pl.pallas_call(
        _tc2_body,
        grid=(_GRID,),
        in_specs=[
            pl.BlockSpec((NC, _BR, H), lambda i: (0, i, 0)),
            pl.BlockSpec((_BR, H), lambda i: (i, 0)),
            pl.BlockSpec((_BR, 1), lambda i: (i, 0)),
            pl.BlockSpec((1, H), lambda i: (0, 0)),
            pl.BlockSpec((H, H), lambda i: (0, 0)),
        ],
        out_specs=pl.BlockSpec((_BR, H), lambda i: (i, 0)),
        out_shape=jax.ShapeDtypeStruct((NP, H), jnp.float32),
    )(agg, hh, dinv, b, W)


def _tc3_body(agg_ref, hh_ref, dinv_ref, s_ref, b_ref,
              w3_ref, b3_ref, wc1_ref, bc1_ref, wc2_ref, bc2_ref,
              u_ref, o_ref):
    a = agg_ref[0] + agg_ref[1]
    dinv = dinv_ref[...]
    h2 = jnp.maximum(dinv * (a + hh_ref[...]) + b_ref[...], 0.0)
    sv = s_ref[...]
    w = dinv * (sv[:, 0:1] + sv[:, 1:2] + dinv)
    rowid = (lax.broadcasted_iota(jnp.int32, (_BR, 1), 0)
             + pl.program_id(0) * _BR)
    w = jnp.where(rowid < N, w, 0.0)

    @pl.when(pl.program_id(0) == 0)
    def _():
        u_ref[...] = jnp.zeros_like(u_ref)

    u_ref[...] += jnp.sum(w * h2, axis=0, keepdims=True)

    @pl.when(pl.program_id(0) == _GRID - 1)
    def _():
        g = jnp.dot(u_ref[...], w3_ref[...],
                    preferred_element_type=jnp.float32) * (1.0 / N) + b3_ref[...]
        z = jnp.maximum(jnp.dot(g, wc1_ref[...],
                                preferred_element_type=jnp.float32)
                        + bc1_ref[...], 0.0)
        o_ref[...] = jnp.dot(z, wc2_ref[...],
                             preferred_element_type=jnp.float32) + bc2_ref[...]


def _tc3(agg, hh, dinv, sT, b, W3, b3, Wc1, bc1, Wc2, bc2):
    full = lambda i: (0, 0)
    return pl.pallas_call(
        _tc3_body,
        grid=(_GRID,),
        in_specs=[
            pl.BlockSpec((NC, _BR, H), lambda i: (0, i, 0)),
            pl.BlockSpec((_BR, H), lambda i: (i, 0)),
            pl.BlockSpec((_BR, 1), lambda i: (i, 0)),
            pl.BlockSpec((_BR, NC), lambda i: (i, 0)),
            pl.BlockSpec((1, H), full),
            pl.BlockSpec((H, H), full),
            pl.BlockSpec((1, H), full),
            pl.BlockSpec((H, H // 2), full),
            pl.BlockSpec((1, H // 2), full),
            pl.BlockSpec((H // 2, 5), full),
            pl.BlockSpec((1, 5), full),
        ],
        out_specs=[
            pl.BlockSpec((1, H), full),
            pl.BlockSpec((1, 5), full),
        ],
        out_shape=[
            jax.ShapeDtypeStruct((1, H), jnp.float32),
            jax.ShapeDtypeStruct((1, 5), jnp.float32),
        ],
    )(agg, hh, dinv, sT, b, W3, b3, Wc1, bc1, Wc2, bc2)


@jax.jit
def kernel(x, edge_index, W1, b1, W2, b2, W3, b3, Wc1, bc1, Wc2, bc2):
    ei = edge_index.astype(jnp.int32)
    pad_idx = N + jnp.arange(NCHP * K - E, dtype=jnp.int32) % (NP - N)
    pad_idx = jnp.broadcast_to(pad_idx, (2, NCHP * K - E))
    ei2 = jnp.concatenate([ei, pad_idx], axis=1).reshape(2, NCHP, K)
    src2, dst2 = ei2[0], ei2[1]
    x_pad = jnp.pad(x, ((0, NP - N), (0, 0)))
    z1 = jnp.zeros((NP,), jnp.float32)
    z2 = jnp.zeros((NP, H), jnp.float32)

    degp = _deg_kernel(dst2, z1)
    hh1p, dinv = _tc1(degp.T, x_pad, W1)
    agg1, sp = _edge_pass_s(hh1p, src2, dst2, dinv[:, 0], z2, z1)
    hh2p = _tc2(agg1, hh1p, dinv, b1.reshape(1, H), W2)
    agg2 = _edge_pass(hh2p, src2, dst2, z2)
    _, out = _tc3(agg2, hh2p, dinv, sp.T, b2.reshape(1, H),
                  W3, b3.reshape(1, H), Wc1, bc1.reshape(1, H // 2),
                  Wc2, bc2.reshape(1, 5))
    return out
